# Initial kernel scaffold; baseline (speedup 1.0000x reference)
import jax
import jax.numpy as jnp
from jax.experimental import pallas as pl


def kernel(x, gate_w, gate_b, W1, b1, W2, b2):
    raise NotImplementedError("write your pallas kernel here")


# speedup vs baseline: 1.8994x; 1.8994x over previous
import functools
import jax
import jax.numpy as jnp
from jax.experimental import pallas as pl
from jax.experimental.pallas import tpu as pltpu

_D = 1024
_H = 4096
_E = 8
_K = 2
_N = 2048
_B = 256
_HB = 512
_NHB = _H // _HB
_P = _N * _K
_G = _P // _B + _E
_ROWS = _G * _B

_INTERP = False


def _gate_kernel(x_ref, gw_ref, gb_ref, w_ref, i_ref):
    logits = jnp.dot(x_ref[...], gw_ref[...],
                     preferred_element_type=jnp.float32) + gb_ref[...]
    cols = jax.lax.broadcasted_iota(jnp.int32, logits.shape, 1)
    m1 = jnp.max(logits, axis=1)
    i1 = jnp.argmax(logits, axis=1).astype(jnp.int32)
    masked = jnp.where(cols == i1[:, None], -jnp.inf, logits)
    m2 = jnp.max(masked, axis=1)
    i2 = jnp.argmax(masked, axis=1).astype(jnp.int32)
    e2 = jnp.exp(m2 - m1)
    w1 = 1.0 / (1.0 + e2)
    w2 = e2 / (1.0 + e2)
    w_ref[...] = jnp.stack([w1, w2], axis=1)
    i_ref[...] = jnp.stack([i1, i2], axis=1)


def _gate(x, gate_w, gate_b):
    bt = 256
    return pl.pallas_call(
        _gate_kernel,
        grid=(_N // bt,),
        in_specs=[
            pl.BlockSpec((bt, _D), lambda t: (t, 0)),
            pl.BlockSpec((_D, _E), lambda t: (0, 0)),
            pl.BlockSpec((_E,), lambda t: (0,)),
        ],
        out_specs=[
            pl.BlockSpec((bt, _K), lambda t: (t, 0)),
            pl.BlockSpec((bt, _K), lambda t: (t, 0)),
        ],
        out_shape=[
            jax.ShapeDtypeStruct((_N, _K), jnp.float32),
            jax.ShapeDtypeStruct((_N, _K), jnp.int32),
        ],
        interpret=_INTERP,
    )(x, gate_w, gate_b)


def _route(idx, w):
    idxf = idx.reshape(-1)
    onehot = (idxf[:, None] == jnp.arange(_E)[None, :]).astype(jnp.int32)
    counts = onehot.sum(0)
    pc = ((counts + _B - 1) // _B) * _B
    ends = jnp.cumsum(pc)
    off = ends - pc
    ranks = jnp.cumsum(onehot, 0) - onehot
    r = (ranks * onehot).sum(1)
    pos = off[idxf] + r
    src = jnp.zeros((_ROWS,), jnp.int32).at[pos].set(
        jnp.arange(_P, dtype=jnp.int32) // _K)
    ws = jnp.zeros((_ROWS,), jnp.float32).at[pos].set(w.reshape(-1))
    te = jnp.minimum(
        jnp.searchsorted(ends, jnp.arange(_G, dtype=jnp.int32) * _B,
                         side='right').astype(jnp.int32),
        _E - 1)
    return te, src, ws, pos.reshape(_N, _K)


def _gelu(a):
    return a * 0.5 * (1.0 + jax.lax.erf(a * 0.7071067811865476))


def _ffn_kernel(te_ref, xs_ref, w1_ref, b1_ref, w2_ref, b2_ref, ws_ref,
                out_ref):
    h = pl.program_id(1)
    a = jnp.dot(xs_ref[...], w1_ref[0],
                preferred_element_type=jnp.float32) + b1_ref[0]
    y = jnp.dot(_gelu(a), w2_ref[0], preferred_element_type=jnp.float32)

    @pl.when(h == 0)
    def _():
        out_ref[...] = jnp.zeros_like(out_ref)

    out_ref[...] += y

    @pl.when(h == _NHB - 1)
    def _():
        out_ref[...] = (out_ref[...] + b2_ref[0]) * ws_ref[...]


def _ffn(te, xs, W1, b1, W2, b2, ws):
    grid_spec = pltpu.PrefetchScalarGridSpec(
        num_scalar_prefetch=1,
        grid=(_G, _NHB),
        in_specs=[
            pl.BlockSpec((_B, _D), lambda g, h, te: (g, 0)),
            pl.BlockSpec((1, _D, _HB), lambda g, h, te: (te[g], 0, h)),
            pl.BlockSpec((1, 1, _HB), lambda g, h, te: (te[g], 0, h)),
            pl.BlockSpec((1, _HB, _D), lambda g, h, te: (te[g], h, 0)),
            pl.BlockSpec((1, 1, _D), lambda g, h, te: (te[g], 0, 0)),
            pl.BlockSpec((_B, 1), lambda g, h, te: (g, 0)),
        ],
        out_specs=pl.BlockSpec((_B, _D), lambda g, h, te: (g, 0)),
    )
    return pl.pallas_call(
        _ffn_kernel,
        grid_spec=grid_spec,
        out_shape=jax.ShapeDtypeStruct((_ROWS, _D), jnp.float32),
        interpret=_INTERP,
    )(te, xs, W1, b1.reshape(_E, 1, _H), W2, b2.reshape(_E, 1, _D),
      ws.reshape(_ROWS, 1))


def kernel(x, gate_w, gate_b, W1, b1, W2, b2):
    w, idx = _gate(x, gate_w, gate_b)
    te, src, ws, pos = _route(idx, w)
    xs = x[src]
    ys = _ffn(te, xs, W1, b1, W2, b2, ws)
    out = ys[pos[:, 0]] + ys[pos[:, 1]]
    return out
